# baseline (speedup 1.0000x reference)
import dataclasses
import functools

import jax
import jax.numpy as jnp
from jax import lax
from jax.experimental import pallas as pl
from jax.experimental.pallas import tpu as pltpu
from jax.experimental.pallas import tpu_sc as plsc

GAMMA = 12.0
NC = 2
NS = 16
NW = NC * NS
LANES = 16
IDX_CHUNK = 128
SUPER = 128


def _params():
    cp = pltpu.CompilerParams()
    if "needs_layout_passes" in pltpu.CompilerParams.__dataclass_fields__:
        cp = dataclasses.replace(cp, needs_layout_passes=False)
    if "use_tc_tiling_on_sc" in pltpu.CompilerParams.__dataclass_fields__:
        cp = dataclasses.replace(cp, use_tc_tiling_on_sc=True)
    return cp


@functools.lru_cache(maxsize=None)
def _build(B, D):
    assert B % (NW * LANES) == 0 and D % LANES == 0
    bpw = B // NW
    nchunk = bpw // IDX_CHUNK
    assert nchunk == 4
    half = bpw // 2
    hchunk = nchunk // 2

    mesh = plsc.VectorSubcoreMesh(core_axis_name="c", subcore_axis_name="s")

    @functools.partial(
        pl.kernel,
        mesh=mesh,
        compiler_params=_params(),
        out_type=jax.ShapeDtypeStruct((B,), jnp.float32),
        scratch_types=[
            pltpu.VMEM((24, IDX_CHUNK), jnp.int32),
            pltpu.VMEM((half, SUPER), jnp.float32),
            pltpu.VMEM((half, SUPER), jnp.float32),
            pltpu.VMEM((half, SUPER), jnp.float32),
            pltpu.VMEM((LANES, LANES), jnp.float32),
            pltpu.VMEM((bpw,), jnp.float32),
            pltpu.SemaphoreType.DMA,
        ],
    )
    def kge(ent_hbm, rel_hbm, ids_hbm, out_hbm,
            ix_v, h_v, r_v, t_v, p_v, s_v, sem):
        wid = lax.axis_index("s") * NC + lax.axis_index("c")
        pltpu.sync_copy(ids_hbm.at[wid], ix_v)

        iota16 = lax.iota(jnp.int32, 16)

        for p in range(2):
            for k in range(hchunk):
                kk = p * hchunk + k
                dst = pl.ds(k * IDX_CHUNK, IDX_CHUNK)
                pltpu.async_copy(ent_hbm.at[ix_v.at[kk]], h_v.at[dst], sem)
                pltpu.async_copy(rel_hbm.at[ix_v.at[nchunk + kk]],
                                 r_v.at[dst], sem)
                pltpu.async_copy(ent_hbm.at[ix_v.at[2 * nchunk + kk]],
                                 t_v.at[dst], sem)

            pltpu.make_async_copy(ent_hbm.at[pl.ds(0, half)], h_v, sem).wait()
            pltpu.make_async_copy(rel_hbm.at[pl.ds(0, half)], r_v, sem).wait()
            pltpu.make_async_copy(ent_hbm.at[pl.ds(0, half)], t_v, sem).wait()

            for k in range(hchunk):
                kk = p * hchunk + k

                @pl.loop(0, IDX_CHUNK, step=LANES)
                def _grp(l0, k=k, kk=kk):
                    ohv = ix_v[3 * nchunk + kk, pl.ds(l0, LANES)]
                    orv = ix_v[4 * nchunk + kk, pl.ds(l0, LANES)]
                    otv = ix_v[5 * nchunk + kk, pl.ds(l0, LANES)]
                    for j in range(LANES):
                        b = k * IDX_CHUNK + l0 + j
                        oh, orr, ot = ohv[j], orv[j], otv[j]
                        acc = jnp.abs(h_v[b, pl.ds(oh, LANES)]
                                      + r_v[b, pl.ds(orr, LANES)]
                                      - t_v[b, pl.ds(ot, LANES)])
                        for c in range(1, D // LANES):
                            acc = acc + jnp.abs(
                                h_v[b, pl.ds(oh + c * LANES, LANES)]
                                + r_v[b, pl.ds(orr + c * LANES, LANES)]
                                - t_v[b, pl.ds(ot + c * LANES, LANES)])
                        p_v[j, :] = acc

                    tot = plsc.load_gather(
                        p_v, [iota16, jnp.full((16,), 0, jnp.int32)])
                    for j in range(1, LANES):
                        tot = tot + plsc.load_gather(
                            p_v, [iota16, jnp.full((16,), j, jnp.int32)])
                    s_v[pl.ds(kk * IDX_CHUNK + l0, LANES)] = GAMMA - tot

        pltpu.sync_copy(s_v, out_hbm.at[pl.ds(wid * bpw, bpw)])

    return kge


def kernel(sample, entity_embedding, relation_embedding):
    B = sample.shape[0]
    D = entity_embedding.shape[1]
    ids = sample.astype(jnp.int32)
    bpw = B // NW
    nchunk = bpw // IDX_CHUNK
    ent2 = entity_embedding.reshape(-1, 2 * D)
    rel2 = relation_embedding.reshape(-1, 2 * D)
    sup = ids >> 1
    off = (ids & 1) * D
    blk = jnp.concatenate(
        [sup[:, 0].reshape(NW, nchunk, IDX_CHUNK),
         sup[:, 1].reshape(NW, nchunk, IDX_CHUNK),
         sup[:, 2].reshape(NW, nchunk, IDX_CHUNK),
         off[:, 0].reshape(NW, nchunk, IDX_CHUNK),
         off[:, 1].reshape(NW, nchunk, IDX_CHUNK),
         off[:, 2].reshape(NW, nchunk, IDX_CHUNK)], axis=1)
    score = _build(B, D)(ent2, rel2, blk)
    return score.reshape(B, 1)


# speedup vs baseline: 1.5752x; 1.5752x over previous
import dataclasses
import functools

import jax
import jax.numpy as jnp
from jax import lax
from jax.experimental import pallas as pl
from jax.experimental.pallas import tpu as pltpu
from jax.experimental.pallas import tpu_sc as plsc

GAMMA = 12.0
NC = 2
NS = 16
NW = NC * NS
LANES = 16
IDX_CHUNK = 128


@functools.lru_cache(maxsize=None)
def _build(B, D):
    assert B % (NW * LANES) == 0 and D % LANES == 0
    bpw = B // NW
    nchunk = bpw // IDX_CHUNK
    assert nchunk == 4

    mesh = plsc.VectorSubcoreMesh(core_axis_name="c", subcore_axis_name="s")

    cp = pltpu.CompilerParams()
    if "needs_layout_passes" in pltpu.CompilerParams.__dataclass_fields__:
        cp = dataclasses.replace(cp, needs_layout_passes=False)
    if "use_tc_tiling_on_sc" in pltpu.CompilerParams.__dataclass_fields__:
        cp = dataclasses.replace(cp, use_tc_tiling_on_sc=True)

    @functools.partial(
        pl.kernel,
        mesh=mesh,
        compiler_params=cp,
        out_type=jax.ShapeDtypeStruct((B,), jnp.float32),
        scratch_types=[
            pltpu.VMEM((16, IDX_CHUNK), jnp.int32),
            pltpu.VMEM((bpw // 2, D), jnp.float32),
            pltpu.VMEM((bpw // 2, D), jnp.float32),
            pltpu.VMEM((bpw // 2, D), jnp.float32),
            pltpu.VMEM((LANES, LANES), jnp.float32),
            pltpu.VMEM((bpw,), jnp.float32),
            pltpu.SemaphoreType.DMA,
        ],
    )
    def kge(ent_hbm, rel_hbm, ids_hbm, out_hbm,
            ix_v, h_v, r_v, t_v, p_v, s_v, sem):
        wid = lax.axis_index("s") * NC + lax.axis_index("c")
        pltpu.sync_copy(ids_hbm.at[wid], ix_v)

        iota16 = lax.iota(jnp.int32, 16)
        half = bpw // 2
        hchunk = nchunk // 2

        for p in range(2):
            @pl.loop(0, IDX_CHUNK, step=LANES)
            def _fire(l0):
                for k in range(hchunk):
                    kk = p * hchunk + k
                    hv = ix_v[kk, pl.ds(l0, LANES)]
                    rv = ix_v[nchunk + kk, pl.ds(l0, LANES)]
                    tv = ix_v[2 * nchunk + kk, pl.ds(l0, LANES)]
                    for j in range(LANES):
                        row = k * IDX_CHUNK + l0 + j
                        pltpu.async_copy(ent_hbm.at[hv[j]], h_v.at[row], sem)
                        pltpu.async_copy(rel_hbm.at[rv[j]], r_v.at[row], sem)
                        pltpu.async_copy(ent_hbm.at[tv[j]], t_v.at[row], sem)

            pltpu.make_async_copy(ent_hbm.at[pl.ds(0, half)], h_v, sem).wait()
            pltpu.make_async_copy(rel_hbm.at[pl.ds(0, half)], r_v, sem).wait()
            pltpu.make_async_copy(ent_hbm.at[pl.ds(0, half)], t_v, sem).wait()

            @pl.loop(0, half, step=LANES)
            def _group(g):
                @pl.loop(0, LANES)
                def _row(i):
                    b = g + i
                    acc = jnp.abs(h_v[b, pl.ds(0, LANES)]
                                  + r_v[b, pl.ds(0, LANES)]
                                  - t_v[b, pl.ds(0, LANES)])
                    for c in range(1, D // LANES):
                        sl = pl.ds(c * LANES, LANES)
                        acc = acc + jnp.abs(h_v[b, sl] + r_v[b, sl] - t_v[b, sl])
                    p_v[i, :] = acc

                tot = plsc.load_gather(p_v, [iota16, jnp.full((16,), 0, jnp.int32)])
                for j in range(1, LANES):
                    tot = tot + plsc.load_gather(p_v, [iota16, jnp.full((16,), j, jnp.int32)])
                s_v[pl.ds(p * half + g, LANES)] = GAMMA - tot

        pltpu.sync_copy(s_v, out_hbm.at[pl.ds(wid * bpw, bpw)])

    return kge


def kernel(sample, entity_embedding, relation_embedding):
    B = sample.shape[0]
    D = entity_embedding.shape[1]
    ids = sample.astype(jnp.int32)
    bpw = B // NW
    nchunk = bpw // IDX_CHUNK
    blk = jnp.concatenate(
        [ids[:, 0].reshape(NW, nchunk, IDX_CHUNK),
         ids[:, 1].reshape(NW, nchunk, IDX_CHUNK),
         ids[:, 2].reshape(NW, nchunk, IDX_CHUNK),
         jnp.zeros((NW, 16 - 3 * nchunk, IDX_CHUNK), jnp.int32)], axis=1)
    score = _build(B, D)(entity_embedding, relation_embedding, blk)
    return score.reshape(B, 1)
